# baseline (speedup 1.0000x reference)
import functools
import math

import jax
import jax.numpy as jnp
from jax.experimental import pallas as pl
from jax.experimental.pallas import tpu as pltpu

B, L, D = 1024, 200, 128
VE, VA, VV = 64, 32, 16
TB = 32
N = TB * L


def _embed_body(e_ref, a_ref, v_ref, d_ref, wcat_ref, o_ref, tt_ref, pe_ref):
    lane1 = jax.lax.broadcasted_iota(jnp.int32, (1, D), 1)
    j2 = ((lane1 // 2) * 2).astype(jnp.float32)
    div = jnp.exp(j2 * (-math.log(10000.0) / D))
    phase = (lane1 % 2).astype(jnp.float32) * (math.pi / 2.0)

    @pl.when(pl.program_id(0) == 0)
    def _build_tables():
        h = jax.lax.broadcasted_iota(jnp.int32, (64, D), 0).astype(jnp.float32)
        arg_a = (h * 64.0) * div
        arg_b = h * div + phase
        top = jnp.concatenate(
            [jnp.sin(arg_a), jnp.cos(arg_a), jnp.zeros((64, 2 * D), jnp.float32)],
            axis=1)
        bot = jnp.concatenate(
            [jnp.zeros((64, 2 * D), jnp.float32), jnp.cos(arg_b), jnp.sin(arg_b)],
            axis=1)
        tt_ref[...] = jnp.concatenate([top, bot], axis=0).astype(jnp.bfloat16)
        pos = jax.lax.broadcasted_iota(jnp.int32, (L, D), 0).astype(jnp.float32)
        pe_ref[...] = jnp.sin(pos * div + phase)

    e = e_ref[...][..., None]
    a = (a_ref[...] + VE)[..., None]
    v = (v_ref[...] + (VE + VA))[..., None]
    d = d_ref[...]
    hi = (d >> 6)[..., None]
    lo = ((d & 63) + 64)[..., None]
    lane = jax.lax.broadcasted_iota(jnp.int32, (TB, L, D), 2)
    m = ((lane == e) | (lane == a) | (lane == v)
         ).astype(jnp.bfloat16).reshape(N, D)
    eav = jnp.dot(m, wcat_ref[...].astype(jnp.bfloat16),
                  preferred_element_type=jnp.float32)

    ohl = ((lane == hi) | (lane == lo)).astype(jnp.bfloat16).reshape(N, D)
    r = jnp.dot(ohl, tt_ref[...], preferred_element_type=jnp.float32)
    time_emb = (r[:, 0:D] * r[:, 2 * D:3 * D]
                + r[:, D:2 * D] * r[:, 3 * D:4 * D])

    out = (eav + time_emb).reshape(TB, L, D) + pe_ref[...][None, :, :]
    o_ref[...] = out


@functools.partial(jax.jit, static_argnames=("interpret",))
def kernel(entity_id, attribute_id, value_id, days_since_tpx,
           W_entity, W_attribute, W_value, interpret=False):
    wcat = jnp.zeros((D, D), jnp.float32)
    wcat = jax.lax.dynamic_update_slice(wcat, W_entity, (0, 0))
    wcat = jax.lax.dynamic_update_slice(wcat, W_attribute, (VE, 0))
    wcat = jax.lax.dynamic_update_slice(wcat, W_value, (VE + VA, 0))

    grid = (B // TB,)
    idx_spec = pl.BlockSpec((TB, L), lambda i: (i, 0))
    out = pl.pallas_call(
        _embed_body,
        grid=grid,
        in_specs=[idx_spec, idx_spec, idx_spec, idx_spec,
                  pl.BlockSpec((D, D), lambda i: (0, 0))],
        out_specs=pl.BlockSpec((TB, L, D), lambda i: (i, 0, 0)),
        out_shape=jax.ShapeDtypeStruct((B, L, D), jnp.float32),
        scratch_shapes=[pltpu.VMEM((D, 4 * D), jnp.bfloat16),
                        pltpu.VMEM((L, D), jnp.float32)],
        compiler_params=pltpu.CompilerParams(
            dimension_semantics=("arbitrary",),
        ),
        interpret=interpret,
    )(entity_id, attribute_id, value_id, days_since_tpx, wcat)
    return out


# speedup vs baseline: 1.5184x; 1.5184x over previous
import functools
import math

import jax
import jax.numpy as jnp
from jax.experimental import pallas as pl
from jax.experimental.pallas import tpu as pltpu

B, L, D = 1024, 200, 128
VE, VA, VV = 64, 32, 16
TB = 32
N = TB * L


def _embed_body(e_ref, a_ref, v_ref, d_ref, wcat_ref, o_ref, tt_ref, pe_ref):
    lane1 = jax.lax.broadcasted_iota(jnp.int32, (1, D), 1)
    j2 = ((lane1 // 2) * 2).astype(jnp.float32)
    div = jnp.exp(j2 * (-math.log(10000.0) / D))
    phase = (lane1 % 2).astype(jnp.float32) * (math.pi / 2.0)

    @pl.when(pl.program_id(0) == 0)
    def _build_tables():
        h = jax.lax.broadcasted_iota(jnp.int32, (64, D), 0).astype(jnp.float32)
        arg_a = (h * 64.0) * div
        arg_b = h * div + phase
        top = jnp.concatenate(
            [jnp.sin(arg_a), jnp.cos(arg_a), jnp.zeros((64, 2 * D), jnp.float32)],
            axis=1)
        bot = jnp.concatenate(
            [jnp.zeros((64, 2 * D), jnp.float32), jnp.cos(arg_b), jnp.sin(arg_b)],
            axis=1)
        tt_ref[...] = jnp.concatenate([top, bot], axis=0).astype(jnp.bfloat16)
        pos = jax.lax.broadcasted_iota(jnp.int32, (L, D), 0).astype(jnp.float32)
        pe_ref[...] = jnp.sin(pos * div + phase)

    e = e_ref[...].astype(jnp.int16)[..., None]
    a = a_ref[...].astype(jnp.int16)[..., None]
    v = v_ref[...].astype(jnp.int16)[..., None]
    d = d_ref[...][..., None]
    lane16 = jax.lax.broadcasted_iota(jnp.int16, (TB, L, D), 2)
    m = ((lane16 == e) | (lane16 == a + VE) | (lane16 == v + (VE + VA))
         ).astype(jnp.bfloat16).reshape(N, D)
    eav = jnp.dot(m, wcat_ref[...].astype(jnp.bfloat16),
                  preferred_element_type=jnp.float32)

    lane = jax.lax.broadcasted_iota(jnp.int32, (TB, L, D), 2)
    hi = d >> 6
    lo = d & 63
    ohl = ((lane == hi) | (lane == lo + 64)).astype(jnp.bfloat16).reshape(N, D)
    r = jnp.dot(ohl, tt_ref[...], preferred_element_type=jnp.float32)
    time_emb = (r[:, 0:D] * r[:, 2 * D:3 * D]
                + r[:, D:2 * D] * r[:, 3 * D:4 * D])

    out = (eav + time_emb).reshape(TB, L, D) + pe_ref[...][None, :, :]
    o_ref[...] = out


@functools.partial(jax.jit, static_argnames=("interpret",))
def kernel(entity_id, attribute_id, value_id, days_since_tpx,
           W_entity, W_attribute, W_value, interpret=False):
    wcat = jnp.zeros((D, D), jnp.float32)
    wcat = jax.lax.dynamic_update_slice(wcat, W_entity, (0, 0))
    wcat = jax.lax.dynamic_update_slice(wcat, W_attribute, (VE, 0))
    wcat = jax.lax.dynamic_update_slice(wcat, W_value, (VE + VA, 0))

    grid = (B // TB,)
    idx_spec = pl.BlockSpec((TB, L), lambda i: (i, 0))
    out = pl.pallas_call(
        _embed_body,
        grid=grid,
        in_specs=[idx_spec, idx_spec, idx_spec, idx_spec,
                  pl.BlockSpec((D, D), lambda i: (0, 0))],
        out_specs=pl.BlockSpec((TB, L, D), lambda i: (i, 0, 0)),
        out_shape=jax.ShapeDtypeStruct((B, L, D), jnp.float32),
        scratch_shapes=[pltpu.VMEM((D, 4 * D), jnp.bfloat16),
                        pltpu.VMEM((L, D), jnp.float32)],
        compiler_params=pltpu.CompilerParams(
            dimension_semantics=("arbitrary",),
        ),
        interpret=interpret,
    )(entity_id, attribute_id, value_id, days_since_tpx, wcat)
    return out


# speedup vs baseline: 1.6033x; 1.0559x over previous
import functools
import math

import jax
import jax.numpy as jnp
from jax.experimental import pallas as pl
from jax.experimental.pallas import tpu as pltpu

B, L, D = 1024, 200, 128
VE, VA, VV = 64, 32, 16
TB = 64
N = TB * L
H = D // 2


def _embed_body(e_ref, a_ref, v_ref, d_ref, wcat_ref, o_ref, tt_ref, pe_ref):
    lane1 = jax.lax.broadcasted_iota(jnp.int32, (1, D), 1)
    j2 = ((lane1 // 2) * 2).astype(jnp.float32)
    div = jnp.exp(j2 * (-math.log(10000.0) / D))
    phase = (lane1 % 2).astype(jnp.float32) * (math.pi / 2.0)

    @pl.when(pl.program_id(0) == 0)
    def _build_tables():
        h = jax.lax.broadcasted_iota(jnp.int32, (64, D), 0).astype(jnp.float32)
        arg_a = (h * 64.0) * div
        arg_b = h * div + phase
        top = jnp.concatenate(
            [jnp.sin(arg_a), jnp.cos(arg_a), jnp.zeros((64, 2 * D), jnp.float32)],
            axis=1)
        bot = jnp.concatenate(
            [jnp.zeros((64, 2 * D), jnp.float32), jnp.cos(arg_b), jnp.sin(arg_b)],
            axis=1)
        tt_ref[...] = jnp.concatenate([top, bot], axis=0).astype(jnp.bfloat16)
        pos = jax.lax.broadcasted_iota(jnp.int32, (L, D), 0).astype(jnp.float32)
        pe_ref[...] = jnp.sin(pos * div + phase)

    e = e_ref[...].astype(jnp.int16)[..., None]
    a = a_ref[...].astype(jnp.int16)[..., None]
    v = v_ref[...].astype(jnp.int16)[..., None]
    d = d_ref[...][..., None]
    lane16 = jax.lax.broadcasted_iota(jnp.int16, (TB, L, D), 2)
    m = ((lane16 == e) | (lane16 == a + VE) | (lane16 == v + (VE + VA))
         ).astype(jnp.bfloat16).reshape(N, D)
    eav = jnp.dot(m, wcat_ref[...].astype(jnp.bfloat16),
                  preferred_element_type=jnp.float32)

    lane = jax.lax.broadcasted_iota(jnp.int32, (TB, L, D), 2)
    hi = d >> 6
    lo = d & 63
    ohl = ((lane == hi) | (lane == lo + 64)).astype(jnp.bfloat16).reshape(N, D)
    r = jnp.dot(ohl, tt_ref[...], preferred_element_type=jnp.float32)
    time_emb = (r[:, 0:D] * r[:, 2 * D:3 * D]
                + r[:, D:2 * D] * r[:, 3 * D:4 * D])

    out = (eav + time_emb).reshape(TB, L, D) + pe_ref[...][None, :, :]
    o_ref[...] = out


@functools.partial(jax.jit, static_argnames=("interpret",))
def kernel(entity_id, attribute_id, value_id, days_since_tpx,
           W_entity, W_attribute, W_value, interpret=False):
    wcat = jnp.zeros((D, D), jnp.float32)
    wcat = jax.lax.dynamic_update_slice(wcat, W_entity, (0, 0))
    wcat = jax.lax.dynamic_update_slice(wcat, W_attribute, (VE, 0))
    wcat = jax.lax.dynamic_update_slice(wcat, W_value, (VE + VA, 0))

    grid = (B // TB,)
    idx_spec = pl.BlockSpec((TB, L), lambda i: (i, 0))
    out = pl.pallas_call(
        _embed_body,
        grid=grid,
        in_specs=[idx_spec, idx_spec, idx_spec, idx_spec,
                  pl.BlockSpec((D, D), lambda i: (0, 0))],
        out_specs=pl.BlockSpec((TB, L, D), lambda i: (i, 0, 0)),
        out_shape=jax.ShapeDtypeStruct((B, L, D), jnp.float32),
        scratch_shapes=[pltpu.VMEM((D, 4 * D), jnp.bfloat16),
                        pltpu.VMEM((L, D), jnp.float32)],
        compiler_params=pltpu.CompilerParams(
            dimension_semantics=("arbitrary",),
        ),
        interpret=interpret,
    )(entity_id, attribute_id, value_id, days_since_tpx, wcat)
    return out


# speedup vs baseline: 1.6093x; 1.0037x over previous
import functools
import math

import jax
import jax.numpy as jnp
from jax.experimental import pallas as pl
from jax.experimental.pallas import tpu as pltpu

B, L, D = 1024, 200, 128
VE, VA, VV = 64, 32, 16
TB = 128
N = TB * L
H = D // 2


def _embed_body(e_ref, a_ref, v_ref, d_ref, wcat_ref, o_ref, tt_ref, pe_ref):
    lane1 = jax.lax.broadcasted_iota(jnp.int32, (1, D), 1)
    j2 = ((lane1 // 2) * 2).astype(jnp.float32)
    div = jnp.exp(j2 * (-math.log(10000.0) / D))
    phase = (lane1 % 2).astype(jnp.float32) * (math.pi / 2.0)

    @pl.when(pl.program_id(0) == 0)
    def _build_tables():
        h = jax.lax.broadcasted_iota(jnp.int32, (64, D), 0).astype(jnp.float32)
        arg_a = (h * 64.0) * div
        arg_b = h * div + phase
        top = jnp.concatenate(
            [jnp.sin(arg_a), jnp.cos(arg_a), jnp.zeros((64, 2 * D), jnp.float32)],
            axis=1)
        bot = jnp.concatenate(
            [jnp.zeros((64, 2 * D), jnp.float32), jnp.cos(arg_b), jnp.sin(arg_b)],
            axis=1)
        tt_ref[...] = jnp.concatenate([top, bot], axis=0).astype(jnp.bfloat16)
        pos = jax.lax.broadcasted_iota(jnp.int32, (L, D), 0).astype(jnp.float32)
        pe_ref[...] = jnp.sin(pos * div + phase)

    e = e_ref[...].astype(jnp.int16)[..., None]
    a = a_ref[...].astype(jnp.int16)[..., None]
    v = v_ref[...].astype(jnp.int16)[..., None]
    d = d_ref[...][..., None]
    lane16 = jax.lax.broadcasted_iota(jnp.int16, (TB, L, D), 2)
    m = ((lane16 == e) | (lane16 == a + VE) | (lane16 == v + (VE + VA))
         ).astype(jnp.bfloat16).reshape(N, D)
    eav = jnp.dot(m, wcat_ref[...].astype(jnp.bfloat16),
                  preferred_element_type=jnp.float32)

    lane = jax.lax.broadcasted_iota(jnp.int32, (TB, L, D), 2)
    hi = d >> 6
    lo = d & 63
    ohl = ((lane == hi) | (lane == lo + 64)).astype(jnp.bfloat16).reshape(N, D)
    r = jnp.dot(ohl, tt_ref[...], preferred_element_type=jnp.float32)
    time_emb = (r[:, 0:D] * r[:, 2 * D:3 * D]
                + r[:, D:2 * D] * r[:, 3 * D:4 * D])

    out = (eav + time_emb).reshape(TB, L, D) + pe_ref[...][None, :, :]
    o_ref[...] = out


@functools.partial(jax.jit, static_argnames=("interpret",))
def kernel(entity_id, attribute_id, value_id, days_since_tpx,
           W_entity, W_attribute, W_value, interpret=False):
    wcat = jnp.zeros((D, D), jnp.float32)
    wcat = jax.lax.dynamic_update_slice(wcat, W_entity, (0, 0))
    wcat = jax.lax.dynamic_update_slice(wcat, W_attribute, (VE, 0))
    wcat = jax.lax.dynamic_update_slice(wcat, W_value, (VE + VA, 0))

    grid = (B // TB,)
    idx_spec = pl.BlockSpec((TB, L), lambda i: (i, 0))
    out = pl.pallas_call(
        _embed_body,
        grid=grid,
        in_specs=[idx_spec, idx_spec, idx_spec, idx_spec,
                  pl.BlockSpec((D, D), lambda i: (0, 0))],
        out_specs=pl.BlockSpec((TB, L, D), lambda i: (i, 0, 0)),
        out_shape=jax.ShapeDtypeStruct((B, L, D), jnp.float32),
        scratch_shapes=[pltpu.VMEM((D, 4 * D), jnp.bfloat16),
                        pltpu.VMEM((L, D), jnp.float32)],
        compiler_params=pltpu.CompilerParams(
            dimension_semantics=("arbitrary",),
        ),
        interpret=interpret,
    )(entity_id, attribute_id, value_id, days_since_tpx, wcat)
    return out
